# Initial kernel scaffold; baseline (speedup 1.0000x reference)
import jax
import jax.numpy as jnp
from jax import lax
from jax.experimental import pallas as pl
from jax.experimental.pallas import tpu as pltpu
from jax.experimental.pallas import tpu_sc as plsc

B, L, V, D = 4096, 50, 1000000, 32
NC, NS = 2, 16
NW = NC * NS
BW = B // NW
CB = 16
G = BW // CB
RPC = CB * L
NSUB = 8
SUB = RPC // NSUB
HALF = 16


def _body(pred, c1, c2, c3, c4, c5, table, out,
          idx0, idx1, rows0, rows1, predbuf, outbuf,
          rsem0, rsem1, isem0, isem1):
    idx_refs = (pred, c1, c2, c3, c4, c5)
    idx_v = (idx0, idx1)
    rows = (rows0, rows1)
    rsem = (rsem0, rsem1)
    isem = (isem0, isem1)
    wid = lax.axis_index("s") * NC + lax.axis_index("c")

    def issue_idx(a, g, p):
        pltpu.async_copy(idx_refs[a].at[wid, g], idx_v[p], isem[p])

    def wait_idx(a, g, p):
        pltpu.make_async_copy(idx_refs[a].at[wid, g], idx_v[p], isem[p]).wait()

    def issue_rows(p):
        for i in range(NSUB):
            pltpu.async_copy(table.at[idx_v[p].at[i]],
                             rows[p].at[pl.ds(i * SUB, SUB)], rsem[p])

    def drain_rows(p):
        pltpu.make_async_copy(table.at[pl.ds(0, RPC)], rows[p], rsem[p]).wait()

    def loop_g(g, carry):
        best = [None] * CB
        bi = [None] * CB
        for a in range(6):
            sp = a % 2
            npar = (a + 1) % 2
            if a < 5:
                wait_idx(a + 1, g, npar)
            else:
                @pl.when(g < G - 1)
                def _():
                    wait_idx(0, g + 1, npar)
            drain_rows(sp)
            if a < 5:
                issue_rows(npar)
            else:
                @pl.when(g < G - 1)
                def _():
                    issue_rows(npar)
            if a < 4:
                issue_idx(a + 2, g, sp)
            else:
                @pl.when(g < G - 1)
                def _():
                    issue_idx((a + 2) % 6, g + 1, sp)
            rp = rows[sp]
            zero = jnp.zeros((HALF,), jnp.float32)
            for r in range(CB):
                def lbody(l, c, r=r):
                    a0, a1 = c
                    j = r * L + l
                    return (a0 + rp[j, pl.ds(0, HALF)],
                            a1 + rp[j, pl.ds(HALF, HALF)])
                a0, a1 = lax.fori_loop(0, L, lbody, (zero, zero), unroll=5)
                if a == 0:
                    predbuf[r, pl.ds(0, HALF)] = a0
                    predbuf[r, pl.ds(HALF, HALF)] = a1
                else:
                    p0 = predbuf[r, pl.ds(0, HALF)]
                    p1 = predbuf[r, pl.ds(HALF, HALF)]
                    s = jnp.sum(a0 * p0 + a1 * p1)
                    if a == 1:
                        best[r] = s
                        bi[r] = jnp.int32(0)
                    else:
                        upd = s > best[r]
                        best[r] = jnp.where(upd, s, best[r])
                        bi[r] = jnp.where(upd, jnp.int32(a - 1), bi[r])
        for r in range(CB):
            outbuf[g, r] = bi[r]
        return carry

    pltpu.sync_copy(idx_refs[0].at[wid, 0], idx_v[0])
    issue_rows(0)
    issue_idx(1, 0, 1)
    lax.fori_loop(0, G, loop_g, 0)
    pltpu.sync_copy(outbuf, out.at[wid])


def kernel(pred, c1, c2, c3, c4, c5, table):
    rs = lambda x: x.reshape(NW, G, NSUB, SUB)
    fn = pl.kernel(
        _body,
        out_type=jax.ShapeDtypeStruct((NW, G, CB), jnp.int32),
        mesh=plsc.VectorSubcoreMesh(core_axis_name="c", subcore_axis_name="s",
                                    num_cores=NC, num_subcores=NS),
        scratch_types=[
            pltpu.VMEM((NSUB, SUB), jnp.int32),
            pltpu.VMEM((NSUB, SUB), jnp.int32),
            pltpu.VMEM((RPC, D), jnp.float32),
            pltpu.VMEM((RPC, D), jnp.float32),
            pltpu.VMEM((CB, D), jnp.float32),
            pltpu.VMEM((G, CB), jnp.int32),
            pltpu.SemaphoreType.DMA,
            pltpu.SemaphoreType.DMA,
            pltpu.SemaphoreType.DMA,
            pltpu.SemaphoreType.DMA,
        ],
    )
    out = fn(rs(pred), rs(c1), rs(c2), rs(c3), rs(c4), rs(c5), table)
    return out.reshape(B)


# speedup vs baseline: 3.1124x; 3.1124x over previous
import jax
import jax.numpy as jnp
from jax import lax
from jax.experimental import pallas as pl
from jax.experimental.pallas import tpu as pltpu
from jax.experimental.pallas import tpu_sc as plsc

B, L, V, D = 4096, 50, 1000000, 32
NC, NS = 2, 16
NW = NC * NS
BW = B // NW
CB = 16
G = BW // CB
RPC = CB * L
NSUB = 8
SUB = RPC // NSUB
HALF = 16


def _body(pred, c1, c2, c3, c4, c5, table, out,
          idx0, idx1, rows0, rows1, predbuf, prodbuf, outbuf,
          rsem0, rsem1, isem0, isem1):
    idx_refs = (pred, c1, c2, c3, c4, c5)
    idx_v = (idx0, idx1)
    rows = (rows0, rows1)
    rsem = (rsem0, rsem1)
    isem = (isem0, isem1)
    wid = lax.axis_index("s") * NC + lax.axis_index("c")

    def issue_idx(a, g, p):
        pltpu.async_copy(idx_refs[a].at[wid, g], idx_v[p], isem[p])

    def wait_idx(a, g, p):
        pltpu.make_async_copy(idx_refs[a].at[wid, g], idx_v[p], isem[p]).wait()

    def issue_rows(p):
        for i in range(NSUB):
            pltpu.async_copy(table.at[idx_v[p].at[i]],
                             rows[p].at[pl.ds(i * SUB, SUB)], rsem[p])

    def drain_rows(p):
        pltpu.make_async_copy(table.at[pl.ds(0, RPC)], rows[p], rsem[p]).wait()

    def loop_g(g, carry):
        best = None
        bi = None
        for a in range(6):
            sp = a % 2
            npar = (a + 1) % 2
            if a < 5:
                wait_idx(a + 1, g, npar)
            else:
                @pl.when(g < G - 1)
                def _():
                    wait_idx(0, g + 1, npar)
            drain_rows(sp)
            if a < 5:
                issue_rows(npar)
            else:
                @pl.when(g < G - 1)
                def _():
                    issue_rows(npar)
            if a < 4:
                issue_idx(a + 2, g, sp)
            else:
                @pl.when(g < G - 1)
                def _():
                    issue_idx((a + 2) % 6, g + 1, sp)
            rp = rows[sp]
            zero = jnp.zeros((HALF,), jnp.float32)
            for r in range(CB):
                def lbody(l, c, r=r):
                    a0, a1 = c
                    j = r * L + l
                    return (a0 + rp[j, pl.ds(0, HALF)],
                            a1 + rp[j, pl.ds(HALF, HALF)])
                a0, a1 = lax.fori_loop(0, L, lbody, (zero, zero), unroll=5)
                if a == 0:
                    predbuf[r, pl.ds(0, HALF)] = a0
                    predbuf[r, pl.ds(HALF, HALF)] = a1
                else:
                    p0 = predbuf[r, pl.ds(0, HALF)]
                    p1 = predbuf[r, pl.ds(HALF, HALF)]
                    prodbuf[pl.ds(r * D, HALF)] = a0 * p0
                    prodbuf[pl.ds(r * D + HALF, HALF)] = a1 * p1
            if a >= 1:
                base = lax.iota(jnp.int32, CB) * D
                sv = jnp.zeros((CB,), jnp.float32)
                for d in range(D):
                    sv = sv + plsc.load_gather(prodbuf, [base + d])
                if a == 1:
                    best = sv
                    bi = jnp.zeros((CB,), jnp.int32)
                else:
                    upd = sv > best
                    best = jnp.where(upd, sv, best)
                    bi = jnp.where(upd, jnp.int32(a - 1), bi)
        outbuf[g] = bi
        return carry

    pltpu.sync_copy(idx_refs[0].at[wid, 0], idx_v[0])
    issue_rows(0)
    issue_idx(1, 0, 1)
    lax.fori_loop(0, G, loop_g, 0)
    pltpu.sync_copy(outbuf, out.at[wid])


def kernel(pred, c1, c2, c3, c4, c5, table):
    rs = lambda x: x.reshape(NW, G, NSUB, SUB)
    fn = pl.kernel(
        _body,
        out_type=jax.ShapeDtypeStruct((NW, G, CB), jnp.int32),
        mesh=plsc.VectorSubcoreMesh(core_axis_name="c", subcore_axis_name="s",
                                    num_cores=NC, num_subcores=NS),
        compiler_params=pltpu.CompilerParams(needs_layout_passes=False,
                                             use_tc_tiling_on_sc=False),
        scratch_types=[
            pltpu.VMEM((NSUB, SUB), jnp.int32),
            pltpu.VMEM((NSUB, SUB), jnp.int32),
            pltpu.VMEM((RPC, D), jnp.float32),
            pltpu.VMEM((RPC, D), jnp.float32),
            pltpu.VMEM((CB, D), jnp.float32),
            pltpu.VMEM((CB * D,), jnp.float32),
            pltpu.VMEM((G, CB), jnp.int32),
            pltpu.SemaphoreType.DMA,
            pltpu.SemaphoreType.DMA,
            pltpu.SemaphoreType.DMA,
            pltpu.SemaphoreType.DMA,
        ],
    )
    out = fn(rs(pred), rs(c1), rs(c2), rs(c3), rs(c4), rs(c5), table)
    return out.reshape(B)
